# baseline (speedup 1.0000x reference)
import functools
import jax
import jax.numpy as jnp
from jax import lax
from jax.experimental import pallas as pl
from jax.experimental.pallas import tpu as pltpu, tpu_sc as plsc

V_DIM = 32
NC, NS = 2, 16
NW = NC * NS
NBUF = 2


@functools.lru_cache(maxsize=None)
def _make_sc_kernel(B, L, DEG_ROWS):
    rows_per_w = B // NW
    splits = [(0, 128), (128, L - 128)]
    mesh = plsc.VectorSubcoreMesh(core_axis_name="c", subcore_axis_name="s")

    @functools.partial(
        pl.kernel,
        out_type=jax.ShapeDtypeStruct((B, L, V_DIM), jnp.float32),
        mesh=mesh,
        scratch_types=[
            pltpu.VMEM((rows_per_w, L), jnp.int32),
            pltpu.VMEM((rows_per_w, L), jnp.int32),
            pltpu.VMEM((rows_per_w, L), jnp.int32),
            pltpu.VMEM((NBUF, L, V_DIM), jnp.float32),
            pltpu.VMEM((DEG_ROWS, V_DIM), jnp.float32),
            [pltpu.SemaphoreType.DMA] * NBUF,
            [pltpu.SemaphoreType.DMA] * NBUF,
        ],
        compiler_params=pltpu.CompilerParams(use_tc_tiling_on_sc=False),
    )
    def k(vidx_hbm, pos_hbm, deg_hbm, Wv, Wp, Wd, out_hbm,
          iv, ip, idg, rows, deg_tab, sa, sw):
        wid = lax.axis_index("s") * NC + lax.axis_index("c")
        base = wid * rows_per_w
        pltpu.sync_copy(Wd, deg_tab)
        pltpu.sync_copy(vidx_hbm.at[pl.ds(base, rows_per_w)], iv)
        pltpu.sync_copy(pos_hbm.at[pl.ds(base, rows_per_w)], ip)
        pltpu.sync_copy(deg_hbm.at[pl.ds(base, rows_per_w)], idg)

        def deg_fill(s, p):
            buf = rows.at[p]

            def grp(g, carry):
                ixv = idg[s, pl.ds(g * 16, 16)]
                for l in range(16):
                    ix = ixv[l]
                    r = g * 16 + l
                    buf[r, pl.ds(0, 16)] = deg_tab[ix, pl.ds(0, 16)]
                    buf[r, pl.ds(16, 16)] = deg_tab[ix, pl.ds(16, 16)]
                return carry

            lax.fori_loop(0, L // 16, grp, 0)
            ixv = idg[s, pl.ds(L - 16, 16)]
            for l in range(16):
                ix = ixv[l]
                r = L - 16 + l
                buf[r, pl.ds(0, 16)] = deg_tab[ix, pl.ds(0, 16)]
                buf[r, pl.ds(16, 16)] = deg_tab[ix, pl.ds(16, 16)]

        def fire_adds(s, p):
            buf = rows.at[p]
            for (o, n) in splits:
                dst = buf.at[pl.ds(o, n)]
                pltpu.async_copy(Wv.at[iv.at[s, pl.ds(o, n)]], dst, sa[p],
                                 add=True)
                pltpu.async_copy(Wp.at[ip.at[s, pl.ds(o, n)]], dst, sa[p],
                                 add=True)

        def wait_adds(s, p):
            buf = rows.at[p]
            for (o, n) in splits:
                dst = buf.at[pl.ds(o, n)]
                pltpu.make_async_copy(
                    Wv.at[iv.at[s, pl.ds(o, n)]], dst, sa[p]).wait()
                pltpu.make_async_copy(
                    Wp.at[ip.at[s, pl.ds(o, n)]], dst, sa[p]).wait()

        def fire_wb(s, p):
            pltpu.async_copy(rows.at[p], out_hbm.at[base + s], sw[p])

        def wait_wb(p):
            pltpu.make_async_copy(rows.at[p], out_hbm.at[base], sw[p]).wait()

        def step(s, u):
            p = u % NBUF
            q = (u + NBUF - 1) % NBUF
            @pl.when(s >= NBUF)
            def _():
                wait_wb(p)

            deg_fill(s, p)
            fire_adds(s, p)
            @pl.when(s >= 1)
            def _():
                wait_adds(s - 1, q)
                fire_wb(s - 1, q)

        def round_(g, carry):
            for u in range(NBUF):
                step(g * NBUF + u, u)
            return carry

        lax.fori_loop(0, rows_per_w // NBUF, round_, 0)
        p_last = (rows_per_w - 1) % NBUF
        wait_adds(rows_per_w - 1, p_last)
        fire_wb(rows_per_w - 1, p_last)
        for p in range(NBUF):
            wait_wb(p)

    return k


def kernel(vidx, pos, deg, W_vidx, W_pos, W_deg):
    B, L = vidx.shape
    return _make_sc_kernel(B, L, W_deg.shape[0])(
        vidx, pos, deg, W_vidx, W_pos, W_deg)
